# baseline (speedup 1.0000x reference)
import functools
import math

import jax
import jax.numpy as jnp
from jax import lax
from jax.experimental import pallas as pl
from jax.experimental.pallas import tpu as pltpu
from jax.experimental.pallas import tpu_sc as plsc

IN_DIM = 34
OUT_DIM = 256
HALF = OUT_DIM // 2
SEQ = 2048
BATCH = 128
N_EMB = 1001

CHUNK_ROWS = (256, 512, 512, 512, 256)
CHUNK_OFFS = (0, 256, 768, 1280, 1792)
K_CHUNKS = len(CHUNK_ROWS)

NC = 2
NS = 16
NW = NC * NS
NBUF = 4

_SC_MESH = plsc.VectorSubcoreMesh(
    core_axis_name="c", subcore_axis_name="s", num_cores=NC, num_subcores=NS
)


def _make_sc_gather(k):
    cseq = CHUNK_ROWS[k]
    rows_w = cseq // NW
    ngrp = rows_w // NBUF

    @functools.partial(
        pl.kernel,
        out_type=jax.ShapeDtypeStruct((cseq, BATCH, HALF), jnp.int32),
        mesh=_SC_MESH,
        scratch_types=[
            pltpu.VMEM((rows_w, BATCH), jnp.int32),
            pltpu.VMEM((NBUF, BATCH, HALF), jnp.int32),
            pltpu.SemaphoreType.DMA((NBUF,)),
            pltpu.SemaphoreType.DMA((NBUF,)),
        ],
    )
    def sc_gather(table_hbm, ids_hbm, out_hbm, idx_v, rows_v, sem_g, sem_s):
        wid = lax.axis_index("s") * NC + lax.axis_index("c")
        base = wid * rows_w
        src = CHUNK_OFFS[k] + base
        pltpu.sync_copy(ids_hbm.at[pl.ds(src, rows_w)], idx_v)

        for b in range(NBUF):
            pltpu.async_copy(table_hbm.at[idx_v.at[b]], rows_v.at[b], sem_g.at[b])

        def group(g, carry):
            for b in range(NBUF):
                j = g * NBUF + b
                pltpu.make_async_copy(
                    table_hbm.at[idx_v.at[j]], rows_v.at[b], sem_g.at[b]
                ).wait()
                pltpu.async_copy(rows_v.at[b], out_hbm.at[base + j], sem_s.at[b])

            @pl.when(g + 1 < ngrp)
            def _():
                for b in range(NBUF):
                    j = g * NBUF + b
                    pltpu.make_async_copy(
                        rows_v.at[b], out_hbm.at[base + j], sem_s.at[b]
                    ).wait()
                    jn = (g + 1) * NBUF + b
                    pltpu.async_copy(
                        table_hbm.at[idx_v.at[jn]], rows_v.at[b], sem_g.at[b]
                    )

            return carry

        lax.fori_loop(0, ngrp, group, 0, unroll=False)

        last = ngrp - 1
        for b in range(NBUF):
            j = last * NBUF + b
            pltpu.make_async_copy(
                rows_v.at[b], out_hbm.at[base + j], sem_s.at[b]
            ).wait()

    return sc_gather


S_BLK = 128


def _tc_body(x_ref, g_ref, w_ref, b_ref, o_ref):
    xb = x_ref[...].reshape(IN_DIM, S_BLK * BATCH)
    acc = lax.dot_general(
        xb, w_ref[...], (((0,), (0,)), ((), ())),
        preferred_element_type=jnp.float32,
    )
    acc = acc + b_ref[...]
    g = g_ref[...].reshape(S_BLK * BATCH, HALF)
    lo = lax.bitcast_convert_type(g << 16, jnp.float32)
    hi = lax.bitcast_convert_type(g & jnp.int32(-65536), jnp.float32)
    out = jnp.concatenate([acc[:, :HALF] + lo, acc[:, HALF:] + hi], axis=-1)
    o_ref[...] = out.reshape(S_BLK, BATCH, OUT_DIM)


def _tc_fused_chunk(k, x, gathered_c, w, b, prev=None):
    blk_off = CHUNK_OFFS[k] // S_BLK
    in_specs = [
        pl.BlockSpec((IN_DIM, S_BLK, BATCH), lambda i: (0, blk_off + i, 0)),
        pl.BlockSpec((S_BLK, BATCH, HALF), lambda i: (i, 0, 0)),
        pl.BlockSpec((IN_DIM, OUT_DIM), lambda i: (0, 0)),
        pl.BlockSpec((1, OUT_DIM), lambda i: (0, 0)),
    ]
    args = [x, gathered_c, w, b]
    aliases = {}
    body = _tc_body
    if prev is not None:
        in_specs.append(pl.BlockSpec(memory_space=pl.ANY))
        args.append(prev)
        aliases = {4: 0}

        def body(x_ref, g_ref, w_ref, b_ref, prev_ref, o_ref):
            del prev_ref
            _tc_body(x_ref, g_ref, w_ref, b_ref, o_ref)

    return pl.pallas_call(
        body,
        grid=(CHUNK_ROWS[k] // S_BLK,),
        in_specs=in_specs,
        out_specs=pl.BlockSpec(
            (S_BLK, BATCH, OUT_DIM), lambda i: (blk_off + i, 0, 0)
        ),
        out_shape=jax.ShapeDtypeStruct((SEQ, BATCH, OUT_DIM), jnp.float32),
        input_output_aliases=aliases,
        compiler_params=pltpu.CompilerParams(
            dimension_semantics=("arbitrary",),
        ),
    )(*args)


def kernel(x, emb_table, W_epoch, W_cfg, b_cfg):
    ids = x[..., 0].astype(jnp.int32)
    inv_std = math.sqrt(12.0)
    w = jnp.concatenate(
        [jnp.zeros((OUT_DIM, 1), jnp.float32), W_epoch * inv_std, W_cfg], axis=1
    ).T
    b = (b_cfg - 0.5 * inv_std * W_epoch[:, 0]).reshape(1, OUT_DIM)
    em = emb_table.astype(jnp.bfloat16)
    packed = lax.bitcast_convert_type(
        jnp.stack([em[:, :HALF], em[:, HALF:]], axis=-1), jnp.int32
    )

    xt = jnp.transpose(x, (2, 0, 1))
    gathered = [_make_sc_gather(k)(packed, ids) for k in range(K_CHUNKS)]
    out = None
    for k in range(K_CHUNKS):
        out = _tc_fused_chunk(k, xt, gathered[k], w, b, out)
    return out


# speedup vs baseline: 1.0220x; 1.0220x over previous
import functools
import math

import jax
import jax.numpy as jnp
from jax import lax
from jax.experimental import pallas as pl
from jax.experimental.pallas import tpu as pltpu
from jax.experimental.pallas import tpu_sc as plsc

IN_DIM = 34
OUT_DIM = 256
HALF = OUT_DIM // 2
SEQ = 2048
BATCH = 128
N_EMB = 1001

CHUNK_ROWS = (256, 768, 768, 256)
CHUNK_OFFS = (0, 256, 1024, 1792)
K_CHUNKS = len(CHUNK_ROWS)

NC = 2
NS = 16
NW = NC * NS
NBUF = 4

_SC_MESH = plsc.VectorSubcoreMesh(
    core_axis_name="c", subcore_axis_name="s", num_cores=NC, num_subcores=NS
)


def _make_sc_gather(k):
    cseq = CHUNK_ROWS[k]
    rows_w = cseq // NW
    ngrp = rows_w // NBUF

    @functools.partial(
        pl.kernel,
        out_type=jax.ShapeDtypeStruct((cseq, BATCH, HALF), jnp.int32),
        mesh=_SC_MESH,
        scratch_types=[
            pltpu.VMEM((rows_w, BATCH), jnp.int32),
            pltpu.VMEM((NBUF, BATCH, HALF), jnp.int32),
            pltpu.SemaphoreType.DMA((NBUF,)),
            pltpu.SemaphoreType.DMA((NBUF,)),
        ],
    )
    def sc_gather(table_hbm, ids_hbm, out_hbm, idx_v, rows_v, sem_g, sem_s):
        wid = lax.axis_index("s") * NC + lax.axis_index("c")
        base = wid * rows_w
        src = CHUNK_OFFS[k] + base
        pltpu.sync_copy(ids_hbm.at[pl.ds(src, rows_w)], idx_v)

        for b in range(NBUF):
            pltpu.async_copy(table_hbm.at[idx_v.at[b]], rows_v.at[b], sem_g.at[b])

        def group(g, carry):
            for b in range(NBUF):
                j = g * NBUF + b
                pltpu.make_async_copy(
                    table_hbm.at[idx_v.at[j]], rows_v.at[b], sem_g.at[b]
                ).wait()
                pltpu.async_copy(rows_v.at[b], out_hbm.at[base + j], sem_s.at[b])

            @pl.when(g + 1 < ngrp)
            def _():
                for b in range(NBUF):
                    j = g * NBUF + b
                    pltpu.make_async_copy(
                        rows_v.at[b], out_hbm.at[base + j], sem_s.at[b]
                    ).wait()
                    jn = (g + 1) * NBUF + b
                    pltpu.async_copy(
                        table_hbm.at[idx_v.at[jn]], rows_v.at[b], sem_g.at[b]
                    )

            return carry

        lax.fori_loop(0, ngrp, group, 0, unroll=False)

        last = ngrp - 1
        for b in range(NBUF):
            j = last * NBUF + b
            pltpu.make_async_copy(
                rows_v.at[b], out_hbm.at[base + j], sem_s.at[b]
            ).wait()

    return sc_gather


S_BLK = 128


def _tc_body(x_ref, g_ref, w_ref, b_ref, o_ref):
    xb = x_ref[...].reshape(IN_DIM, S_BLK * BATCH)
    acc = lax.dot_general(
        xb, w_ref[...], (((0,), (0,)), ((), ())),
        preferred_element_type=jnp.float32,
    )
    acc = acc + b_ref[...]
    g = g_ref[...].reshape(S_BLK * BATCH, HALF)
    lo = lax.bitcast_convert_type(g << 16, jnp.float32)
    hi = lax.bitcast_convert_type(g & jnp.int32(-65536), jnp.float32)
    out = jnp.concatenate([acc[:, :HALF] + lo, acc[:, HALF:] + hi], axis=-1)
    o_ref[...] = out.reshape(S_BLK, BATCH, OUT_DIM)


def _tc_fused_chunk(k, x, gathered_c, w, b, prev=None):
    blk_off = CHUNK_OFFS[k] // S_BLK
    in_specs = [
        pl.BlockSpec((IN_DIM, S_BLK, BATCH), lambda i: (0, blk_off + i, 0)),
        pl.BlockSpec((S_BLK, BATCH, HALF), lambda i: (i, 0, 0)),
        pl.BlockSpec((IN_DIM, OUT_DIM), lambda i: (0, 0)),
        pl.BlockSpec((1, OUT_DIM), lambda i: (0, 0)),
    ]
    args = [x, gathered_c, w, b]
    aliases = {}
    body = _tc_body
    if prev is not None:
        in_specs.append(pl.BlockSpec(memory_space=pl.ANY))
        args.append(prev)
        aliases = {4: 0}

        def body(x_ref, g_ref, w_ref, b_ref, prev_ref, o_ref):
            del prev_ref
            _tc_body(x_ref, g_ref, w_ref, b_ref, o_ref)

    return pl.pallas_call(
        body,
        grid=(CHUNK_ROWS[k] // S_BLK,),
        in_specs=in_specs,
        out_specs=pl.BlockSpec(
            (S_BLK, BATCH, OUT_DIM), lambda i: (blk_off + i, 0, 0)
        ),
        out_shape=jax.ShapeDtypeStruct((SEQ, BATCH, OUT_DIM), jnp.float32),
        input_output_aliases=aliases,
        compiler_params=pltpu.CompilerParams(
            dimension_semantics=("arbitrary",),
        ),
    )(*args)


def kernel(x, emb_table, W_epoch, W_cfg, b_cfg):
    ids = x[..., 0].astype(jnp.int32)
    inv_std = math.sqrt(12.0)
    w = jnp.concatenate(
        [jnp.zeros((OUT_DIM, 1), jnp.float32), W_epoch * inv_std, W_cfg], axis=1
    ).T
    b = (b_cfg - 0.5 * inv_std * W_epoch[:, 0]).reshape(1, OUT_DIM)
    em = emb_table.astype(jnp.bfloat16)
    packed = lax.bitcast_convert_type(
        jnp.stack([em[:, :HALF], em[:, HALF:]], axis=-1), jnp.int32
    )

    xt = jnp.transpose(x, (2, 0, 1))
    gathered = [_make_sc_gather(k)(packed, ids) for k in range(K_CHUNKS)]
    out = None
    for k in range(K_CHUNKS):
        out = _tc_fused_chunk(k, xt, gathered[k], w, b, out)
    return out


# speedup vs baseline: 1.0226x; 1.0006x over previous
import functools
import math

import jax
import jax.numpy as jnp
from jax import lax
from jax.experimental import pallas as pl
from jax.experimental.pallas import tpu as pltpu
from jax.experimental.pallas import tpu_sc as plsc

IN_DIM = 34
OUT_DIM = 256
HALF = OUT_DIM // 2
SEQ = 2048
BATCH = 128
N_EMB = 1001

CHUNK_ROWS = (256, 768, 768, 256)
CHUNK_OFFS = (0, 256, 1024, 1792)
K_CHUNKS = len(CHUNK_ROWS)

NC = 2
NS = 16
NW = NC * NS
NBUF = 4

_SC_MESH = plsc.VectorSubcoreMesh(
    core_axis_name="c", subcore_axis_name="s", num_cores=NC, num_subcores=NS
)


def _make_sc_gather(k):
    cseq = CHUNK_ROWS[k]
    rows_w = cseq // NW
    nbuf = 6 if rows_w % 6 == 0 else NBUF
    ngrp = rows_w // nbuf

    @functools.partial(
        pl.kernel,
        out_type=jax.ShapeDtypeStruct((cseq, BATCH, HALF), jnp.int32),
        mesh=_SC_MESH,
        scratch_types=[
            pltpu.VMEM((rows_w, BATCH), jnp.int32),
            pltpu.VMEM((nbuf, BATCH, HALF), jnp.int32),
            pltpu.SemaphoreType.DMA((nbuf,)),
            pltpu.SemaphoreType.DMA((nbuf,)),
        ],
    )
    def sc_gather(table_hbm, ids_hbm, out_hbm, idx_v, rows_v, sem_g, sem_s):
        wid = lax.axis_index("s") * NC + lax.axis_index("c")
        base = wid * rows_w
        src = CHUNK_OFFS[k] + base
        pltpu.sync_copy(ids_hbm.at[pl.ds(src, rows_w)], idx_v)

        for b in range(nbuf):
            pltpu.async_copy(table_hbm.at[idx_v.at[b]], rows_v.at[b], sem_g.at[b])

        def group(g, carry):
            for b in range(nbuf):
                j = g * nbuf + b
                pltpu.make_async_copy(
                    table_hbm.at[idx_v.at[j]], rows_v.at[b], sem_g.at[b]
                ).wait()
                pltpu.async_copy(rows_v.at[b], out_hbm.at[base + j], sem_s.at[b])

            @pl.when(g + 1 < ngrp)
            def _():
                for b in range(nbuf):
                    j = g * nbuf + b
                    pltpu.make_async_copy(
                        rows_v.at[b], out_hbm.at[base + j], sem_s.at[b]
                    ).wait()
                    jn = (g + 1) * nbuf + b
                    pltpu.async_copy(
                        table_hbm.at[idx_v.at[jn]], rows_v.at[b], sem_g.at[b]
                    )

            return carry

        lax.fori_loop(0, ngrp, group, 0, unroll=False)

        last = ngrp - 1
        for b in range(nbuf):
            j = last * nbuf + b
            pltpu.make_async_copy(
                rows_v.at[b], out_hbm.at[base + j], sem_s.at[b]
            ).wait()

    return sc_gather


S_BLK = 128


def _tc_body(x_ref, g_ref, w_ref, b_ref, o_ref):
    xb = x_ref[...].reshape(IN_DIM, S_BLK * BATCH)
    acc = lax.dot_general(
        xb, w_ref[...], (((0,), (0,)), ((), ())),
        preferred_element_type=jnp.float32,
    )
    acc = acc + b_ref[...]
    g = g_ref[...].reshape(S_BLK * BATCH, HALF)
    lo = lax.bitcast_convert_type(g << 16, jnp.float32)
    hi = lax.bitcast_convert_type(g & jnp.int32(-65536), jnp.float32)
    out = jnp.concatenate([acc[:, :HALF] + lo, acc[:, HALF:] + hi], axis=-1)
    o_ref[...] = out.reshape(S_BLK, BATCH, OUT_DIM)


def _tc_fused_chunk(k, x, gathered_c, w, b, prev=None):
    blk_off = CHUNK_OFFS[k] // S_BLK
    in_specs = [
        pl.BlockSpec((IN_DIM, S_BLK, BATCH), lambda i: (0, blk_off + i, 0)),
        pl.BlockSpec((S_BLK, BATCH, HALF), lambda i: (i, 0, 0)),
        pl.BlockSpec((IN_DIM, OUT_DIM), lambda i: (0, 0)),
        pl.BlockSpec((1, OUT_DIM), lambda i: (0, 0)),
    ]
    args = [x, gathered_c, w, b]
    aliases = {}
    body = _tc_body
    if prev is not None:
        in_specs.append(pl.BlockSpec(memory_space=pl.ANY))
        args.append(prev)
        aliases = {4: 0}

        def body(x_ref, g_ref, w_ref, b_ref, prev_ref, o_ref):
            del prev_ref
            _tc_body(x_ref, g_ref, w_ref, b_ref, o_ref)

    return pl.pallas_call(
        body,
        grid=(CHUNK_ROWS[k] // S_BLK,),
        in_specs=in_specs,
        out_specs=pl.BlockSpec(
            (S_BLK, BATCH, OUT_DIM), lambda i: (blk_off + i, 0, 0)
        ),
        out_shape=jax.ShapeDtypeStruct((SEQ, BATCH, OUT_DIM), jnp.float32),
        input_output_aliases=aliases,
        compiler_params=pltpu.CompilerParams(
            dimension_semantics=("arbitrary",),
        ),
    )(*args)


def kernel(x, emb_table, W_epoch, W_cfg, b_cfg):
    ids = x[..., 0].astype(jnp.int32)
    inv_std = math.sqrt(12.0)
    w = jnp.concatenate(
        [jnp.zeros((OUT_DIM, 1), jnp.float32), W_epoch * inv_std, W_cfg], axis=1
    ).T
    b = (b_cfg - 0.5 * inv_std * W_epoch[:, 0]).reshape(1, OUT_DIM)
    em = emb_table.astype(jnp.bfloat16)
    packed = lax.bitcast_convert_type(
        jnp.stack([em[:, :HALF], em[:, HALF:]], axis=-1), jnp.int32
    )

    xt = jnp.transpose(x, (2, 0, 1))
    gathered = [_make_sc_gather(k)(packed, ids) for k in range(K_CHUNKS)]
    out = None
    for k in range(K_CHUNKS):
        out = _tc_fused_chunk(k, xt, gathered[k], w, b, out)
    return out
